# baseline (speedup 1.0000x reference)
import jax
import jax.numpy as jnp
from jax.experimental import pallas as pl
from jax.experimental.pallas import tpu as pltpu

B, S, D = 1, 2048, 768
H, HD = 8, 64
QC = 256
IH, IHD = 2, 32
TOPK = 8
P, L = 256, 16
WIN = 64
FFM = 4
NEG = -1e9
BS = 256

_f32 = jnp.float32
_bf16 = jnp.bfloat16


def _dot(a, b):
    return jnp.dot(a, b, preferred_element_type=_f32)


def _bdot(a, b):
    return jnp.dot(a.astype(_bf16), b.astype(_bf16),
                   preferred_element_type=_f32)


def _bdot_nt(a, b):
    return jax.lax.dot_general(a.astype(_bf16), b.astype(_bf16),
                               (((1,), (1,)), ((), ())),
                               preferred_element_type=_f32)


def _rms_in(x, w):
    return x * jax.lax.rsqrt(jnp.mean(x * x, axis=-1, keepdims=True) + 1e-6) * w


def _proj_kernel(x_ref, nw_ref, wq1_ref, wq2_ref, wk_ref, wv_ref, wqi_ref,
                 h_ref, q_ref, k_ref, v_ref, qi_ref):
    x = x_ref[...]
    h = _rms_in(x, nw_ref[...])
    h_ref[...] = h
    hb = h.astype(_bf16)
    q_ref[...] = _bdot(jnp.dot(hb, wq1_ref[...], preferred_element_type=_f32),
                       wq2_ref[...]).astype(_bf16)
    k_ref[...] = jnp.dot(hb, wk_ref[...], preferred_element_type=_f32).astype(_bf16)
    v_ref[...] = jnp.dot(hb, wv_ref[...], preferred_element_type=_f32).astype(_bf16)
    qi_ref[...] = _dot(h, wqi_ref[...])


def _mega_kernel(qi_ref, end_ref, mask_ref, q_ref, kp_ref, kc_ref, vp_ref,
                 vc_ref, h_ref, idx_ref, wk_ref, wv_ref, wki_ref, skT_ref,
                 sv_ref, x_ref, wo_ref, nw2_ref, w1_ref, w2_ref,
                 o_ref, pk_s, pv_s, ki_s):
    i = pl.program_id(0)
    scale = HD ** -0.5

    @pl.when(i == 0)
    def _():
        idx = idx_ref[...]
        iota = jax.lax.broadcasted_iota(jnp.int32, (P, S), 1)
        acc = jnp.zeros((P, S), _f32)
        for l in range(L):
            acc += (idx[:, l:l + 1] == iota).astype(_f32)
        ph = _dot(acc, h_ref[...]) * (1.0 / L)
        phb = ph.astype(_bf16)
        pk_s[...] = jnp.dot(phb, wk_ref[...], preferred_element_type=_f32).astype(_bf16)
        pv_s[...] = jnp.dot(phb, wv_ref[...], preferred_element_type=_f32).astype(_bf16)
        ki_s[...] = _dot(ph, wki_ref[...])

    qi = qi_ref[...]
    scores = jax.lax.dot_general(qi, ki_s[...], (((1,), (1,)), ((), ())),
                                 preferred_element_type=_f32)
    pos = i * BS + jax.lax.broadcasted_iota(jnp.int32, (BS, P), 0)
    allowed = (end_ref[...] <= pos) & (mask_ref[...] != 0)
    scores = jnp.where(allowed, scores, NEG)
    r = scores
    thr = jnp.full((BS, 1), NEG, _f32)
    for _ in range(TOPK):
        thr = jnp.max(r, axis=-1, keepdims=True)
        r = jnp.where(r >= thr, NEG, r)
    bias = jnp.where(allowed & (scores >= thr), 0.0, NEG)

    q = q_ref[...]
    kp, kc = kp_ref[...], kc_ref[...]
    vp, vc = vp_ref[...], vc_ref[...]
    pk, pv = pk_s[...], pv_s[...]

    srow = jax.lax.broadcasted_iota(jnp.int32, (BS, BS), 0)
    tcol = jax.lax.broadcasted_iota(jnp.int32, (BS, BS), 1)
    mask_p = (BS + srow - tcol < WIN) & (i > 0)
    dist_c = srow - tcol
    mask_c = (dist_c >= 0) & (dist_c < WIN)

    ctxs = []
    for h in range(H):
        sl = slice(h * HD, (h + 1) * HD)
        qh = q[:, sl]
        lp = jnp.where(mask_p, _bdot_nt(qh, kp[:, sl]) * scale, NEG)
        lc = jnp.where(mask_c, _bdot_nt(qh, kc[:, sl]) * scale, NEG)
        lph = _bdot_nt(qh, pk[:, sl]) * scale + bias
        ls = _bdot(qh, skT_ref[:, h:h + 1]) * scale
        m = jnp.maximum(jnp.maximum(jnp.max(lp, -1, keepdims=True),
                                    jnp.max(lc, -1, keepdims=True)),
                        jnp.maximum(jnp.max(lph, -1, keepdims=True), ls))
        ep = jnp.exp(lp - m)
        ec = jnp.exp(lc - m)
        eph = jnp.exp(lph - m)
        es = jnp.exp(ls - m)
        denom = (jnp.sum(ep, -1, keepdims=True) + jnp.sum(ec, -1, keepdims=True)
                 + jnp.sum(eph, -1, keepdims=True) + es)
        ctx = (_bdot(ep, vp[:, sl]) + _bdot(ec, vc[:, sl]) + _bdot(eph, pv[:, sl])
               + es * sv_ref[h:h + 1, :])
        ctxs.append(ctx / denom)
    ctx_all = jnp.concatenate(ctxs, axis=1)
    x2 = x_ref[...] + jnp.dot(ctx_all.astype(_bf16), wo_ref[...],
                              preferred_element_type=_f32)

    h2 = _rms_in(x2, nw2_ref[...])
    ff = jnp.dot(h2.astype(_bf16), w1_ref[...], preferred_element_type=_f32)
    o_ref[...] = x2 + jnp.dot(jax.nn.gelu(ff).astype(_bf16), w2_ref[...],
                              preferred_element_type=_f32)


def _full(shape):
    n = len(shape)
    return pl.BlockSpec(shape, lambda *a: (0,) * n)


def kernel(x, phrase_mask, phrase_token_idx, phrase_end_pos, phrase_id,
           norm1_w, Wq1, Wq2, Wk, Wv, Wo, Wqi, Wki, sink_k, sink_v,
           norm2_w, W1, W2):
    xs = x.reshape(S, D)
    nw1 = norm1_w.reshape(1, D)
    nw2 = norm2_w.reshape(1, D)
    idx = phrase_token_idx.reshape(P, L).astype(jnp.int32)
    end = phrase_end_pos.reshape(1, P).astype(jnp.int32)
    pmask = phrase_mask.reshape(1, P).astype(jnp.int32)
    wq1b, wq2b = Wq1.astype(_bf16), Wq2.astype(_bf16)
    wkb, wvb, wob = Wk.astype(_bf16), Wv.astype(_bf16), Wo.astype(_bf16)
    w1b, w2b = W1.astype(_bf16), W2.astype(_bf16)
    skTb = sink_k.T.astype(_bf16)

    nblk = S // BS
    seq = lambda i: (i, 0)

    h, q, k, v, qi = pl.pallas_call(
        _proj_kernel,
        grid=(nblk,),
        in_specs=[pl.BlockSpec((BS, D), seq), _full((1, D)),
                  _full((D, QC)), _full((QC, H * HD)), _full((D, H * HD)),
                  _full((D, H * HD)), _full((D, IH * IHD))],
        out_specs=[pl.BlockSpec((BS, D), seq), pl.BlockSpec((BS, H * HD), seq),
                   pl.BlockSpec((BS, H * HD), seq), pl.BlockSpec((BS, H * HD), seq),
                   pl.BlockSpec((BS, IH * IHD), seq)],
        out_shape=[jax.ShapeDtypeStruct((S, D), _f32),
                   jax.ShapeDtypeStruct((S, H * HD), _bf16),
                   jax.ShapeDtypeStruct((S, H * HD), _bf16),
                   jax.ShapeDtypeStruct((S, H * HD), _bf16),
                   jax.ShapeDtypeStruct((S, IH * IHD), _f32)],
    )(xs, nw1, wq1b, wq2b, wkb, wvb, Wqi)

    prev = lambda i: (jnp.maximum(i - 1, 0), 0)
    out = pl.pallas_call(
        _mega_kernel,
        grid=(nblk,),
        in_specs=[pl.BlockSpec((BS, IH * IHD), seq),
                  _full((1, P)), _full((1, P)),
                  pl.BlockSpec((BS, H * HD), seq),
                  pl.BlockSpec((BS, H * HD), prev), pl.BlockSpec((BS, H * HD), seq),
                  pl.BlockSpec((BS, H * HD), prev), pl.BlockSpec((BS, H * HD), seq),
                  _full((S, D)), _full((P, L)),
                  _full((D, H * HD)), _full((D, H * HD)), _full((D, IH * IHD)),
                  _full((HD, H)), _full((H, HD)),
                  pl.BlockSpec((BS, D), seq), _full((H * HD, D)),
                  _full((1, D)), _full((D, FFM * D)), _full((FFM * D, D))],
        out_specs=pl.BlockSpec((BS, D), seq),
        out_shape=jax.ShapeDtypeStruct((S, D), _f32),
        scratch_shapes=[pltpu.VMEM((P, H * HD), _bf16),
                        pltpu.VMEM((P, H * HD), _bf16),
                        pltpu.VMEM((P, IH * IHD), _f32)],
    )(qi, end, pmask, q, k, k, v, v, h, idx, wkb, wvb, Wki, skTb, sink_v,
      xs, wob, nw2, w1b, w2b)

    return out.reshape(B, S, D)


# speedup vs baseline: 1.3340x; 1.3340x over previous
import jax
import jax.numpy as jnp
from jax.experimental import pallas as pl
from jax.experimental.pallas import tpu as pltpu

B, S, D = 1, 2048, 768
H, HD = 8, 64
QC = 256
IH, IHD = 2, 32
TOPK = 8
P, L = 256, 16
WIN = 64
FFM = 4
NEG = -1e9
BS = 256
G = 64
NG = BS // G

_f32 = jnp.float32
_bf16 = jnp.bfloat16


def _dot(a, b):
    return jnp.dot(a, b, preferred_element_type=_f32)


def _bdot_nt(a, b):
    return jax.lax.dot_general(a, b, (((1,), (1,)), ((), ())),
                               preferred_element_type=_f32)


def _rms_in(x, w):
    return x * jax.lax.rsqrt(jnp.mean(x * x, axis=-1, keepdims=True) + 1e-6) * w


def _proj_kernel(x_ref, nw_ref, wq1_ref, wq2_ref, wk_ref, wv_ref, wqi_ref,
                 h_ref, q_ref, k_ref, v_ref, qi_ref,
                 wq1_s, wq2_s, wk_s, wv_s):
    @pl.when(pl.program_id(0) == 0)
    def _():
        wq1_s[...] = wq1_ref[...].astype(_bf16)
        wq2_s[...] = wq2_ref[...].astype(_bf16)
        wk_s[...] = wk_ref[...].astype(_bf16)
        wv_s[...] = wv_ref[...].astype(_bf16)

    x = x_ref[...]
    h = _rms_in(x, nw_ref[...])
    h_ref[...] = h
    hb = h.astype(_bf16)
    q_ref[...] = _dot(_dot(hb, wq1_s[...]).astype(_bf16),
                      wq2_s[...]).astype(_bf16)
    k_ref[...] = _dot(hb, wk_s[...]).astype(_bf16)
    v_ref[...] = _dot(hb, wv_s[...]).astype(_bf16)
    qi_ref[...] = _dot(h, wqi_ref[...])


def _mega_kernel(qi_ref, end_ref, mask_ref, q_ref, kp_ref, kc_ref, vp_ref,
                 vc_ref, h_ref, idx_ref, wk_ref, wv_ref, wki_ref, skT_ref,
                 sv_ref, x_ref, wo_ref, nw2_ref, w1_ref, w2_ref,
                 o_ref, pk_s, pv_s, ki_s, wo_s, w1_s, w2_s):
    i = pl.program_id(0)
    scale = HD ** -0.5

    @pl.when(i == 0)
    def _():
        wo_s[...] = wo_ref[...].astype(_bf16)
        w1_s[...] = w1_ref[...].astype(_bf16)
        w2_s[...] = w2_ref[...].astype(_bf16)
        idx = idx_ref[...]
        iota = jax.lax.broadcasted_iota(jnp.int32, (P, S), 1)
        acc = jnp.zeros((P, S), _f32)
        for l in range(L):
            acc += (idx[:, l:l + 1] == iota).astype(_f32)
        ph = _dot(acc, h_ref[...]) * (1.0 / L)
        phb = ph.astype(_bf16)
        pk_s[...] = _dot(phb, wk_ref[...].astype(_bf16)).astype(_bf16)
        pv_s[...] = _dot(phb, wv_ref[...].astype(_bf16)).astype(_bf16)
        ki_s[...] = _dot(ph, wki_ref[...])

    qi = qi_ref[...]
    scores = _bdot_nt(qi, ki_s[...])
    pos = i * BS + jax.lax.broadcasted_iota(jnp.int32, (BS, P), 0)
    allowed = (end_ref[...] <= pos) & (mask_ref[...] != 0)
    scores = jnp.where(allowed, scores, NEG)
    r = scores
    thr = jnp.full((BS, 1), NEG, _f32)
    for _ in range(TOPK):
        thr = jnp.max(r, axis=-1, keepdims=True)
        r = jnp.where(r >= thr, NEG, r)
    bias = jnp.where(allowed & (scores >= thr), 0.0, NEG)

    q = q_ref[...]
    kp, kc = kp_ref[...], kc_ref[...]
    vp, vc = vp_ref[...], vc_ref[...]
    pk, pv = pk_s[...], pv_s[...]

    khalo = [jnp.concatenate([kp[BS - G:, :], kc[:G, :]], axis=0)]
    vhalo = [jnp.concatenate([vp[BS - G:, :], vc[:G, :]], axis=0)]
    for g in range(1, NG):
        khalo.append(kc[(g - 1) * G:(g + 1) * G, :])
        vhalo.append(vc[(g - 1) * G:(g + 1) * G, :])
    rrow = jax.lax.broadcasted_iota(jnp.int32, (G, 2 * G), 0)
    ccol = jax.lax.broadcasted_iota(jnp.int32, (G, 2 * G), 1)
    band = (ccol > rrow) & (ccol <= rrow + G)
    wmask = [band & ((ccol >= G) | (i > 0))] + [band] * (NG - 1)

    ctxs = []
    for h in range(H):
        sl = slice(h * HD, (h + 1) * HD)
        qh = q[:, sl]
        lw = jnp.concatenate(
            [jnp.where(wmask[g],
                       _bdot_nt(qh[g * G:(g + 1) * G, :], khalo[g][:, sl])
                       * scale, NEG)
             for g in range(NG)], axis=0)
        lph = _bdot_nt(qh, pk[:, sl]) * scale + bias
        ls = _dot(qh, skT_ref[:, h:h + 1]) * scale
        m = jnp.maximum(jnp.max(lw, -1, keepdims=True),
                        jnp.maximum(jnp.max(lph, -1, keepdims=True), ls))
        ew = jnp.exp(lw - m)
        eph = jnp.exp(lph - m)
        es = jnp.exp(ls - m)
        denom = (jnp.sum(ew, -1, keepdims=True)
                 + jnp.sum(eph, -1, keepdims=True) + es)
        ewb = ew.astype(_bf16)
        ctx = jnp.concatenate(
            [_dot(ewb[g * G:(g + 1) * G, :], vhalo[g][:, sl])
             for g in range(NG)], axis=0)
        ctx = ctx + _dot(eph.astype(_bf16), pv[:, sl]) + es * sv_ref[h:h + 1, :]
        ctxs.append(ctx / denom)
    ctx_all = jnp.concatenate(ctxs, axis=1)
    x2 = x_ref[...] + _dot(ctx_all.astype(_bf16), wo_s[...])

    h2 = _rms_in(x2, nw2_ref[...])
    ff = _dot(h2.astype(_bf16), w1_s[...])
    o_ref[...] = x2 + _dot(jax.nn.gelu(ff).astype(_bf16), w2_s[...])


def _full(shape):
    n = len(shape)
    return pl.BlockSpec(shape, lambda *a: (0,) * n)


def kernel(x, phrase_mask, phrase_token_idx, phrase_end_pos, phrase_id,
           norm1_w, Wq1, Wq2, Wk, Wv, Wo, Wqi, Wki, sink_k, sink_v,
           norm2_w, W1, W2):
    xs = x.reshape(S, D)
    nw1 = norm1_w.reshape(1, D)
    nw2 = norm2_w.reshape(1, D)
    idx = phrase_token_idx.reshape(P, L).astype(jnp.int32)
    end = phrase_end_pos.reshape(1, P).astype(jnp.int32)
    pmask = phrase_mask.reshape(1, P).astype(jnp.int32)
    skTb = sink_k.T.astype(_bf16)

    nblk = S // BS
    seq = lambda i: (i, 0)

    h, q, k, v, qi = pl.pallas_call(
        _proj_kernel,
        grid=(nblk,),
        in_specs=[pl.BlockSpec((BS, D), seq), _full((1, D)),
                  _full((D, QC)), _full((QC, H * HD)), _full((D, H * HD)),
                  _full((D, H * HD)), _full((D, IH * IHD))],
        out_specs=[pl.BlockSpec((BS, D), seq), pl.BlockSpec((BS, H * HD), seq),
                   pl.BlockSpec((BS, H * HD), seq), pl.BlockSpec((BS, H * HD), seq),
                   pl.BlockSpec((BS, IH * IHD), seq)],
        out_shape=[jax.ShapeDtypeStruct((S, D), _f32),
                   jax.ShapeDtypeStruct((S, H * HD), _bf16),
                   jax.ShapeDtypeStruct((S, H * HD), _bf16),
                   jax.ShapeDtypeStruct((S, H * HD), _bf16),
                   jax.ShapeDtypeStruct((S, IH * IHD), _f32)],
        scratch_shapes=[pltpu.VMEM((D, QC), _bf16),
                        pltpu.VMEM((QC, H * HD), _bf16),
                        pltpu.VMEM((D, H * HD), _bf16),
                        pltpu.VMEM((D, H * HD), _bf16)],
    )(xs, nw1, Wq1, Wq2, Wk, Wv, Wqi)

    prev = lambda i: (jnp.maximum(i - 1, 0), 0)
    out = pl.pallas_call(
        _mega_kernel,
        grid=(nblk,),
        in_specs=[pl.BlockSpec((BS, IH * IHD), seq),
                  _full((1, P)), _full((1, P)),
                  pl.BlockSpec((BS, H * HD), seq),
                  pl.BlockSpec((BS, H * HD), prev), pl.BlockSpec((BS, H * HD), seq),
                  pl.BlockSpec((BS, H * HD), prev), pl.BlockSpec((BS, H * HD), seq),
                  _full((S, D)), _full((P, L)),
                  _full((D, H * HD)), _full((D, H * HD)), _full((D, IH * IHD)),
                  _full((HD, H)), _full((H, HD)),
                  pl.BlockSpec((BS, D), seq), _full((H * HD, D)),
                  _full((1, D)), _full((D, FFM * D)), _full((FFM * D, D))],
        out_specs=pl.BlockSpec((BS, D), seq),
        out_shape=jax.ShapeDtypeStruct((S, D), _f32),
        scratch_shapes=[pltpu.VMEM((P, H * HD), _bf16),
                        pltpu.VMEM((P, H * HD), _bf16),
                        pltpu.VMEM((P, IH * IHD), _f32),
                        pltpu.VMEM((H * HD, D), _bf16),
                        pltpu.VMEM((D, FFM * D), _bf16),
                        pltpu.VMEM((FFM * D, D), _bf16)],
    )(qi, end, pmask, q, k, k, v, v, h, idx, Wk, Wv, Wki, skTb, sink_v,
      xs, Wo, nw2, W1, W2)

    return out.reshape(B, S, D)


# speedup vs baseline: 1.3768x; 1.0321x over previous
import jax
import jax.numpy as jnp
from jax.experimental import pallas as pl
from jax.experimental.pallas import tpu as pltpu

B, S, D = 1, 2048, 768
H, HD = 8, 64
QC = 256
IH, IHD = 2, 32
TOPK = 8
P, L = 256, 16
WIN = 64
FFM = 4
NEG = -1e9
BS = 256
G = 64
NG = BS // G
SCALE = HD ** -0.5

_f32 = jnp.float32
_bf16 = jnp.bfloat16


def _dot(a, b):
    return jnp.dot(a, b, preferred_element_type=_f32)


def _bdot_nt(a, b):
    return jax.lax.dot_general(a, b, (((1,), (1,)), ((), ())),
                               preferred_element_type=_f32)


def _rms_in(x, w):
    return x * jax.lax.rsqrt(jnp.mean(x * x, axis=-1, keepdims=True) + 1e-6) * w


def _proj_kernel(x_ref, nw_ref, wq1_ref, wq2_ref, wk_ref, wv_ref, wqi_ref,
                 idx_ref, wki_ref,
                 q_ref, k_ref, v_ref, qi_ref, pk_ref, pv_ref, ki_ref,
                 wq1_s, wq2_s, wk_s, wv_s, ph_s):
    i = pl.program_id(0)
    nblk = pl.num_programs(0)

    @pl.when(i == 0)
    def _():
        wq1_s[...] = wq1_ref[...].astype(_bf16)
        wq2_s[...] = wq2_ref[...].astype(_bf16)
        wk_s[...] = wk_ref[...].astype(_bf16)
        wv_s[...] = wv_ref[...].astype(_bf16)
        ph_s[...] = jnp.zeros((P, D), _f32)

    x = x_ref[...]
    h = _rms_in(x, nw_ref[...])
    hb = h.astype(_bf16)
    q_ref[...] = (_dot(_dot(hb, wq1_s[...]).astype(_bf16), wq2_s[...])
                  * SCALE).astype(_bf16)
    k_ref[...] = _dot(hb, wk_s[...]).astype(_bf16)
    v_ref[...] = _dot(hb, wv_s[...]).astype(_bf16)
    qi_ref[...] = _dot(h, wqi_ref[...])

    idx = idx_ref[...]
    iota = i * BS + jax.lax.broadcasted_iota(jnp.int32, (P, BS), 1)
    acc = jnp.zeros((P, BS), _f32)
    for l in range(L):
        acc += (idx[:, l:l + 1] == iota).astype(_f32)
    ph_s[...] += _dot(acc.astype(_bf16), hb)

    @pl.when(i == nblk - 1)
    def _():
        ph = ph_s[...] * (1.0 / L)
        phb = ph.astype(_bf16)
        pk_ref[...] = _dot(phb, wk_s[...]).astype(_bf16)
        pv_ref[...] = _dot(phb, wv_s[...]).astype(_bf16)
        ki_ref[...] = _dot(ph, wki_ref[...])


def _mega_kernel(qi_ref, end_ref, mask_ref, q_ref, kp_ref, kc_ref, vp_ref,
                 vc_ref, pk_ref, pv_ref, ki_ref, skT_ref, sv_ref, x_ref,
                 wo_ref, nw2_ref, w1_ref, w2_ref,
                 o_ref, wo_s, w1_s, w2_s):
    i = pl.program_id(0)

    @pl.when(i == 0)
    def _():
        wo_s[...] = wo_ref[...].astype(_bf16)
        w1_s[...] = w1_ref[...].astype(_bf16)
        w2_s[...] = w2_ref[...].astype(_bf16)

    qi = qi_ref[...]
    scores = _bdot_nt(qi, ki_ref[...])
    pos = i * BS + jax.lax.broadcasted_iota(jnp.int32, (BS, P), 0)
    allowed = (end_ref[...] <= pos) & (mask_ref[...] != 0)
    scores = jnp.where(allowed, scores, NEG)
    r = scores
    thr = jnp.full((BS, 1), NEG, _f32)
    for _ in range(TOPK):
        thr = jnp.max(r, axis=-1, keepdims=True)
        r = jnp.where(r >= thr, NEG, r)
    bias = jnp.where(allowed & (scores >= thr), 0.0, NEG)

    q = q_ref[...]
    kp, kc = kp_ref[...], kc_ref[...]
    vp, vc = vp_ref[...], vc_ref[...]
    pk, pv = pk_ref[...], pv_ref[...]

    khalo = [jnp.concatenate([kp[BS - G:, :], kc[:G, :]], axis=0)]
    vhalo = [jnp.concatenate([vp[BS - G:, :], vc[:G, :]], axis=0)]
    for g in range(1, NG):
        khalo.append(kc[(g - 1) * G:(g + 1) * G, :])
        vhalo.append(vc[(g - 1) * G:(g + 1) * G, :])
    rrow = jax.lax.broadcasted_iota(jnp.int32, (G, 2 * G), 0)
    ccol = jax.lax.broadcasted_iota(jnp.int32, (G, 2 * G), 1)
    band = (ccol > rrow) & (ccol <= rrow + G)
    wmask = [band & ((ccol >= G) | (i > 0))] + [band] * (NG - 1)

    ctxs = []
    for h in range(H):
        sl = slice(h * HD, (h + 1) * HD)
        qh = q[:, sl]
        lw = jnp.concatenate(
            [jnp.where(wmask[g],
                       _bdot_nt(qh[g * G:(g + 1) * G, :], khalo[g][:, sl]),
                       NEG)
             for g in range(NG)], axis=0)
        lph = _bdot_nt(qh, pk[:, sl]) + bias
        ls = _dot(qh, skT_ref[:, h:h + 1])
        m = jnp.maximum(jnp.max(lw, -1, keepdims=True),
                        jnp.maximum(jnp.max(lph, -1, keepdims=True), ls))
        ew = jnp.exp(lw - m)
        eph = jnp.exp(lph - m)
        es = jnp.exp(ls - m)
        inv = 1.0 / (jnp.sum(ew, -1, keepdims=True)
                     + jnp.sum(eph, -1, keepdims=True) + es)
        ewb = ew.astype(_bf16)
        ctx = jnp.concatenate(
            [_dot(ewb[g * G:(g + 1) * G, :], vhalo[g][:, sl])
             for g in range(NG)], axis=0)
        ctx = ctx + _dot(eph.astype(_bf16), pv[:, sl]) + es * sv_ref[h:h + 1, :]
        ctxs.append(ctx * inv)
    ctx_all = jnp.concatenate(ctxs, axis=1)
    x2 = x_ref[...] + _dot(ctx_all.astype(_bf16), wo_s[...])

    h2 = _rms_in(x2, nw2_ref[...])
    ff = _dot(h2.astype(_bf16), w1_s[...])
    o_ref[...] = x2 + _dot(jax.nn.gelu(ff).astype(_bf16), w2_s[...])


def _full(shape):
    n = len(shape)
    return pl.BlockSpec(shape, lambda *a: (0,) * n)


def kernel(x, phrase_mask, phrase_token_idx, phrase_end_pos, phrase_id,
           norm1_w, Wq1, Wq2, Wk, Wv, Wo, Wqi, Wki, sink_k, sink_v,
           norm2_w, W1, W2):
    xs = x.reshape(S, D)
    nw1 = norm1_w.reshape(1, D)
    nw2 = norm2_w.reshape(1, D)
    idx = phrase_token_idx.reshape(P, L).astype(jnp.int32)
    end = phrase_end_pos.reshape(1, P).astype(jnp.int32)
    pmask = phrase_mask.reshape(1, P).astype(jnp.int32)
    skTb = sink_k.T.astype(_bf16)

    nblk = S // BS
    seq = lambda i: (i, 0)

    q, k, v, qi, pk, pv, ki = pl.pallas_call(
        _proj_kernel,
        grid=(nblk,),
        in_specs=[pl.BlockSpec((BS, D), seq), _full((1, D)),
                  _full((D, QC)), _full((QC, H * HD)), _full((D, H * HD)),
                  _full((D, H * HD)), _full((D, IH * IHD)),
                  _full((P, L)), _full((D, IH * IHD))],
        out_specs=[pl.BlockSpec((BS, H * HD), seq), pl.BlockSpec((BS, H * HD), seq),
                   pl.BlockSpec((BS, H * HD), seq), pl.BlockSpec((BS, IH * IHD), seq),
                   _full((P, H * HD)), _full((P, H * HD)), _full((P, IH * IHD))],
        out_shape=[jax.ShapeDtypeStruct((S, H * HD), _bf16),
                   jax.ShapeDtypeStruct((S, H * HD), _bf16),
                   jax.ShapeDtypeStruct((S, H * HD), _bf16),
                   jax.ShapeDtypeStruct((S, IH * IHD), _f32),
                   jax.ShapeDtypeStruct((P, H * HD), _bf16),
                   jax.ShapeDtypeStruct((P, H * HD), _bf16),
                   jax.ShapeDtypeStruct((P, IH * IHD), _f32)],
        scratch_shapes=[pltpu.VMEM((D, QC), _bf16),
                        pltpu.VMEM((QC, H * HD), _bf16),
                        pltpu.VMEM((D, H * HD), _bf16),
                        pltpu.VMEM((D, H * HD), _bf16),
                        pltpu.VMEM((P, D), _f32)],
    )(xs, nw1, Wq1, Wq2, Wk, Wv, Wqi, idx, Wki)

    prev = lambda i: (jnp.maximum(i - 1, 0), 0)
    out = pl.pallas_call(
        _mega_kernel,
        grid=(nblk,),
        in_specs=[pl.BlockSpec((BS, IH * IHD), seq),
                  _full((1, P)), _full((1, P)),
                  pl.BlockSpec((BS, H * HD), seq),
                  pl.BlockSpec((BS, H * HD), prev), pl.BlockSpec((BS, H * HD), seq),
                  pl.BlockSpec((BS, H * HD), prev), pl.BlockSpec((BS, H * HD), seq),
                  _full((P, H * HD)), _full((P, H * HD)), _full((P, IH * IHD)),
                  _full((HD, H)), _full((H, HD)),
                  pl.BlockSpec((BS, D), seq), _full((H * HD, D)),
                  _full((1, D)), _full((D, FFM * D)), _full((FFM * D, D))],
        out_specs=pl.BlockSpec((BS, D), seq),
        out_shape=jax.ShapeDtypeStruct((S, D), _f32),
        scratch_shapes=[pltpu.VMEM((H * HD, D), _bf16),
                        pltpu.VMEM((D, FFM * D), _bf16),
                        pltpu.VMEM((FFM * D, D), _bf16)],
    )(qi, end, pmask, q, k, k, v, v, pk, pv, ki, skTb, sink_v,
      xs, Wo, nw2, W1, W2)

    return out.reshape(B, S, D)
